# baseline (speedup 1.0000x reference)
import jax
import jax.numpy as jnp
from jax import lax
from jax.experimental import pallas as pl
from jax.experimental.pallas import tpu as pltpu
from jax.experimental.pallas import tpu_sc as plsc

N_FACTORS = 64
PACKED_W = N_FACTORS // 2
BATCH = 16384
NUM_WORKERS = 32
B_PER_W = BATCH // NUM_WORKERS
IDX_CHUNK = 128
N_CHUNKS = B_PER_W // IDX_CHUNK
LANES = 16
N_GROUPS = B_PER_W // LANES


def _body(sess_hbm, aid_hbm, asz_hbm, stbl_hbm, atbl_hbm, out_hbm,
          sidx, aidx, asz_v, srows, arows, out_v,
          sem_in, sem_s, sem_a):
    wid = lax.axis_index("c") * 16 + lax.axis_index("s")

    c_idx = pltpu.async_copy(sess_hbm.at[wid], sidx, sem_in)
    c_aidx = pltpu.async_copy(aid_hbm.at[wid], aidx, sem_in)
    c_asz = pltpu.async_copy(asz_hbm.at[wid], asz_v, sem_in)
    c_idx.wait()
    c_aidx.wait()
    c_asz.wait()

    copies = []
    for j in range(N_CHUNKS):
        copies.append(pltpu.async_copy(
            stbl_hbm.at[sidx.at[j]], srows.at[pl.ds(j * IDX_CHUNK, IDX_CHUNK)],
            sem_s))
        copies.append(pltpu.async_copy(
            atbl_hbm.at[aidx.at[j]], arows.at[pl.ds(j * IDX_CHUNK, IDX_CHUNK)],
            sem_a))
    for c in copies:
        c.wait()

    lane = jnp.arange(LANES, dtype=jnp.int32)

    def group_body(g, _):
        row = g * LANES + lane

        def col_body(j, acc):
            col = jnp.full((LANES,), j, dtype=jnp.int32)
            sp = plsc.load_gather(srows, [row, col])
            ap = plsc.load_gather(arows, [row, col])
            s0, s1 = plsc.unpack(plsc.bitcast(sp, jnp.bfloat16),
                                 format=plsc.PackFormat.INTERLEAVED)
            a0, a1 = plsc.unpack(plsc.bitcast(ap, jnp.bfloat16),
                                 format=plsc.PackFormat.INTERLEAVED)
            return acc + s0 * a0 + s1 * a1

        acc = lax.fori_loop(0, PACKED_W, col_body,
                            jnp.zeros((LANES,), jnp.float32))
        scale = asz_v[pl.ds(g * LANES, LANES)]
        out_v[pl.ds(g * LANES, LANES)] = acc * scale
        return 0

    lax.fori_loop(0, N_GROUPS, group_body, 0)

    pltpu.sync_copy(out_v, out_hbm.at[wid])


def kernel(session, aid, aid_size, session_table, aid_table):
    mesh = plsc.VectorSubcoreMesh(core_axis_name="c", subcore_axis_name="s")
    k = pl.kernel(
        _body,
        out_type=jax.ShapeDtypeStruct((NUM_WORKERS, B_PER_W), jnp.float32),
        mesh=mesh,
        compiler_params=pltpu.CompilerParams(
            needs_layout_passes=False, use_tc_tiling_on_sc=False),
        scratch_types=[
            pltpu.VMEM((N_CHUNKS, IDX_CHUNK), jnp.int32),
            pltpu.VMEM((N_CHUNKS, IDX_CHUNK), jnp.int32),
            pltpu.VMEM((B_PER_W,), jnp.float32),
            pltpu.VMEM((B_PER_W, PACKED_W), jnp.int32),
            pltpu.VMEM((B_PER_W, PACKED_W), jnp.int32),
            pltpu.VMEM((B_PER_W,), jnp.float32),
            pltpu.SemaphoreType.DMA,
            pltpu.SemaphoreType.DMA,
            pltpu.SemaphoreType.DMA,
        ],
    )

    def pack_table(t):
        tb = t.astype(jnp.bfloat16).reshape(t.shape[0], PACKED_W, 2)
        return lax.bitcast_convert_type(tb, jnp.int32)

    sess = session.astype(jnp.int32).reshape(NUM_WORKERS, N_CHUNKS, IDX_CHUNK)
    aidr = aid.astype(jnp.int32).reshape(NUM_WORKERS, N_CHUNKS, IDX_CHUNK)
    aszr = aid_size.reshape(NUM_WORKERS, B_PER_W)
    out = k(sess, aidr, aszr, pack_table(session_table), pack_table(aid_table))
    return out.reshape(BATCH)


# speedup vs baseline: 5.7233x; 5.7233x over previous
import jax
import jax.numpy as jnp
from jax import lax
from jax.experimental import pallas as pl
from jax.experimental.pallas import tpu as pltpu
from jax.experimental.pallas import tpu_sc as plsc

N_FACTORS = 64
BATCH = 16384
NUM_WORKERS = 32
B_PER_W = BATCH // NUM_WORKERS
IDX_CHUNK = 128
N_CHUNKS = B_PER_W // IDX_CHUNK
LANES = 16
N_PASSES = 2
B_PER_PASS = B_PER_W // N_PASSES
GROUPS_PER_PASS = B_PER_PASS // LANES

CB = 4096
HALF = 503808
PACKED_ROWS = HALF


def _pack_body(a_ref, b_ref, o_ref):
    xa = jnp.swapaxes(a_ref[...], 0, 1)
    xb = jnp.swapaxes(b_ref[...], 0, 1)
    o_ref[...] = jnp.concatenate([xa, xb], axis=1)


def _pack(tT):
    return pl.pallas_call(
        _pack_body,
        grid=(HALF // CB,),
        in_specs=[
            pl.BlockSpec((64, CB), lambda i: (0, i)),
            pl.BlockSpec((64, CB), lambda i: (0, jnp.minimum(i + 123, 244))),
        ],
        out_specs=pl.BlockSpec((CB, 128), lambda i: (i, 0)),
        out_shape=jax.ShapeDtypeStruct((PACKED_ROWS, 128), jnp.float32),
    )(tT, tT)


def _body(sess_hbm, aid_hbm, asz_hbm, stbl_hbm, atbl_hbm, out_hbm,
          sidx_o, aidx_o, sidx_p, aidx_p, asz_v, srows, arows, out_v,
          sem_in, sem_s, sem_a):
    wid = lax.axis_index("c") * 16 + lax.axis_index("s")

    c1 = pltpu.async_copy(sess_hbm.at[wid], sidx_o, sem_in)
    c2 = pltpu.async_copy(aid_hbm.at[wid], aidx_o, sem_in)
    c3 = pltpu.async_copy(asz_hbm.at[wid], asz_v, sem_in)
    c1.wait()
    c2.wait()
    c3.wait()

    def remap(i, _):
        c = i // 8
        l = (i % 8) * 16
        ov = sidx_o[c, pl.ds(l, 16)]
        sidx_p[c, pl.ds(l, 16)] = ov - jnp.where(
            ov >= HALF, jnp.int32(HALF), jnp.int32(0))
        av = aidx_o[c, pl.ds(l, 16)]
        aidx_p[c, pl.ds(l, 16)] = av - jnp.where(
            av >= HALF, jnp.int32(HALF), jnp.int32(0))
        return 0
    lax.fori_loop(0, N_CHUNKS * 8, remap, 0)

    lane = jnp.arange(LANES, dtype=jnp.int32)

    for p in range(N_PASSES):
        copies = []
        for j in range(2):
            c = p * 2 + j
            copies.append(pltpu.async_copy(
                stbl_hbm.at[sidx_p.at[c]],
                srows.at[pl.ds(j * IDX_CHUNK, IDX_CHUNK)], sem_s))
            copies.append(pltpu.async_copy(
                atbl_hbm.at[aidx_p.at[c]],
                arows.at[pl.ds(j * IDX_CHUNK, IDX_CHUNK)], sem_a))
        for c in copies:
            c.wait()

        def group_body(g, _):
            row = g * LANES + lane
            ch = p * 2 + g // 8
            l = (g % 8) * 16
            so = jnp.where(sidx_o[ch, pl.ds(l, 16)] >= HALF,
                           jnp.int32(N_FACTORS), jnp.int32(0))
            ao = jnp.where(aidx_o[ch, pl.ds(l, 16)] >= HALF,
                           jnp.int32(N_FACTORS), jnp.int32(0))

            def col_body(f, acc):
                sv = plsc.load_gather(srows, [row, so + f])
                av = plsc.load_gather(arows, [row, ao + f])
                return acc + sv * av

            acc = lax.fori_loop(0, N_FACTORS, col_body,
                                jnp.zeros((LANES,), jnp.float32))
            scale = asz_v[pl.ds(p * B_PER_PASS + g * LANES, LANES)]
            out_v[pl.ds(p * B_PER_PASS + g * LANES, LANES)] = acc * scale
            return 0

        lax.fori_loop(0, GROUPS_PER_PASS, group_body, 0)

    pltpu.sync_copy(out_v, out_hbm.at[wid])


def kernel(session, aid, aid_size, session_table, aid_table):
    mesh = plsc.VectorSubcoreMesh(core_axis_name="c", subcore_axis_name="s")
    k = pl.kernel(
        _body,
        out_type=jax.ShapeDtypeStruct((NUM_WORKERS, B_PER_W), jnp.float32),
        mesh=mesh,
        compiler_params=pltpu.CompilerParams(
            needs_layout_passes=False, use_tc_tiling_on_sc=False),
        scratch_types=[
            pltpu.VMEM((N_CHUNKS, IDX_CHUNK), jnp.int32),
            pltpu.VMEM((N_CHUNKS, IDX_CHUNK), jnp.int32),
            pltpu.VMEM((N_CHUNKS, IDX_CHUNK), jnp.int32),
            pltpu.VMEM((N_CHUNKS, IDX_CHUNK), jnp.int32),
            pltpu.VMEM((B_PER_W,), jnp.float32),
            pltpu.VMEM((B_PER_PASS, 2 * N_FACTORS), jnp.float32),
            pltpu.VMEM((B_PER_PASS, 2 * N_FACTORS), jnp.float32),
            pltpu.VMEM((B_PER_W,), jnp.float32),
            pltpu.SemaphoreType.DMA,
            pltpu.SemaphoreType.DMA,
            pltpu.SemaphoreType.DMA,
        ],
    )
    sess = session.astype(jnp.int32).reshape(NUM_WORKERS, N_CHUNKS, IDX_CHUNK)
    aidr = aid.astype(jnp.int32).reshape(NUM_WORKERS, N_CHUNKS, IDX_CHUNK)
    aszr = aid_size.reshape(NUM_WORKERS, B_PER_W)
    ps = _pack(jnp.swapaxes(session_table, 0, 1))
    pa = _pack(jnp.swapaxes(aid_table, 0, 1))
    out = k(sess, aidr, aszr, ps, pa)
    return out.reshape(BATCH)


# speedup vs baseline: 6.4673x; 1.1300x over previous
import jax
import jax.numpy as jnp
from jax import lax
from jax.experimental import pallas as pl
from jax.experimental.pallas import tpu as pltpu
from jax.experimental.pallas import tpu_sc as plsc

N_FACTORS = 64
BATCH = 16384
NUM_WORKERS = 32
B_PER_W = BATCH // NUM_WORKERS
IDX_CHUNK = 128
N_CHUNKS = B_PER_W // IDX_CHUNK
LANES = 16
N_PASSES = 2
B_PER_PASS = B_PER_W // N_PASSES
GROUPS_PER_PASS = B_PER_PASS // LANES

CB = 8192
HALF = 507904
PACKED_ROWS = HALF


def _pack_body(a_ref, b_ref, o_ref):
    xa = jnp.swapaxes(a_ref[...], 0, 1)
    xb = jnp.swapaxes(b_ref[...], 0, 1)
    o_ref[...] = jnp.concatenate([xa, xb], axis=1)


def _pack(tT):
    return pl.pallas_call(
        _pack_body,
        grid=(HALF // CB,),
        in_specs=[
            pl.BlockSpec((64, CB), lambda i: (0, i)),
            pl.BlockSpec((64, CB), lambda i: (0, jnp.minimum(i + 62, 122))),
        ],
        out_specs=pl.BlockSpec((CB, 128), lambda i: (i, 0)),
        out_shape=jax.ShapeDtypeStruct((PACKED_ROWS, 128), jnp.float32),
    )(tT, tT)


def _body(sess_hbm, aid_hbm, asz_hbm, stbl_hbm, atbl_hbm, out_hbm,
          sidx_o, aidx_o, sidx_p, aidx_p, asz_v, srows, arows, out_v,
          sem_in, sem_s, sem_a):
    wid = lax.axis_index("c") * 16 + lax.axis_index("s")

    c1 = pltpu.async_copy(sess_hbm.at[wid], sidx_o, sem_in)
    c2 = pltpu.async_copy(aid_hbm.at[wid], aidx_o, sem_in)
    c3 = pltpu.async_copy(asz_hbm.at[wid], asz_v, sem_in)
    c1.wait()
    c2.wait()
    c3.wait()

    def remap(i, _):
        c = i // 8
        l = (i % 8) * 16
        ov = sidx_o[c, pl.ds(l, 16)]
        sidx_p[c, pl.ds(l, 16)] = ov - jnp.where(
            ov >= HALF, jnp.int32(HALF), jnp.int32(0))
        av = aidx_o[c, pl.ds(l, 16)]
        aidx_p[c, pl.ds(l, 16)] = av - jnp.where(
            av >= HALF, jnp.int32(HALF), jnp.int32(0))
        return 0
    lax.fori_loop(0, N_CHUNKS * 8, remap, 0)

    lane = jnp.arange(LANES, dtype=jnp.int32)

    for p in range(N_PASSES):
        copies = []
        for j in range(2):
            c = p * 2 + j
            copies.append(pltpu.async_copy(
                stbl_hbm.at[sidx_p.at[c]],
                srows.at[pl.ds(j * IDX_CHUNK, IDX_CHUNK)], sem_s))
            copies.append(pltpu.async_copy(
                atbl_hbm.at[aidx_p.at[c]],
                arows.at[pl.ds(j * IDX_CHUNK, IDX_CHUNK)], sem_a))
        for c in copies:
            c.wait()

        def group_body(g, _):
            row = g * LANES + lane
            ch = p * 2 + g // 8
            l = (g % 8) * 16
            so = jnp.where(sidx_o[ch, pl.ds(l, 16)] >= HALF,
                           jnp.int32(N_FACTORS), jnp.int32(0))
            ao = jnp.where(aidx_o[ch, pl.ds(l, 16)] >= HALF,
                           jnp.int32(N_FACTORS), jnp.int32(0))

            def col_body(f, acc):
                sv = plsc.load_gather(srows, [row, so + f])
                av = plsc.load_gather(arows, [row, ao + f])
                return acc + sv * av

            acc = lax.fori_loop(0, N_FACTORS, col_body,
                                jnp.zeros((LANES,), jnp.float32))
            scale = asz_v[pl.ds(p * B_PER_PASS + g * LANES, LANES)]
            out_v[pl.ds(p * B_PER_PASS + g * LANES, LANES)] = acc * scale
            return 0

        lax.fori_loop(0, GROUPS_PER_PASS, group_body, 0)

    pltpu.sync_copy(out_v, out_hbm.at[wid])


def kernel(session, aid, aid_size, session_table, aid_table):
    mesh = plsc.VectorSubcoreMesh(core_axis_name="c", subcore_axis_name="s")
    k = pl.kernel(
        _body,
        out_type=jax.ShapeDtypeStruct((NUM_WORKERS, B_PER_W), jnp.float32),
        mesh=mesh,
        compiler_params=pltpu.CompilerParams(
            needs_layout_passes=False, use_tc_tiling_on_sc=False),
        scratch_types=[
            pltpu.VMEM((N_CHUNKS, IDX_CHUNK), jnp.int32),
            pltpu.VMEM((N_CHUNKS, IDX_CHUNK), jnp.int32),
            pltpu.VMEM((N_CHUNKS, IDX_CHUNK), jnp.int32),
            pltpu.VMEM((N_CHUNKS, IDX_CHUNK), jnp.int32),
            pltpu.VMEM((B_PER_W,), jnp.float32),
            pltpu.VMEM((B_PER_PASS, 2 * N_FACTORS), jnp.float32),
            pltpu.VMEM((B_PER_PASS, 2 * N_FACTORS), jnp.float32),
            pltpu.VMEM((B_PER_W,), jnp.float32),
            pltpu.SemaphoreType.DMA,
            pltpu.SemaphoreType.DMA,
            pltpu.SemaphoreType.DMA,
        ],
    )
    sess = session.astype(jnp.int32).reshape(NUM_WORKERS, N_CHUNKS, IDX_CHUNK)
    aidr = aid.astype(jnp.int32).reshape(NUM_WORKERS, N_CHUNKS, IDX_CHUNK)
    aszr = aid_size.reshape(NUM_WORKERS, B_PER_W)
    ps = _pack(jnp.swapaxes(session_table, 0, 1))
    pa = _pack(jnp.swapaxes(aid_table, 0, 1))
    out = k(sess, aidr, aszr, ps, pa)
    return out.reshape(BATCH)


# speedup vs baseline: 9.0533x; 1.3999x over previous
import jax
import jax.numpy as jnp
from jax import lax
from jax.experimental import pallas as pl
from jax.experimental.pallas import tpu as pltpu
from jax.experimental.pallas import tpu_sc as plsc

N_FACTORS = 64
BATCH = 16384
NUM_WORKERS = 32
B_PER_W = BATCH // NUM_WORKERS
IDX_CHUNK = 128
N_CHUNKS = B_PER_W // IDX_CHUNK
LANES = 16
N_PASSES = 2
B_PER_PASS = B_PER_W // N_PASSES
GROUPS_PER_PASS = B_PER_PASS // LANES

CB = 8192
HALF = 507904
Q = HALF // 2
N_BLOCKS = 123


def _pack_body(a_ref, b_ref, o_ref):
    xa = jnp.swapaxes(a_ref[...].astype(jnp.bfloat16), 0, 1)
    xb = jnp.swapaxes(b_ref[...].astype(jnp.bfloat16), 0, 1)
    pa = pltpu.bitcast(xa, jnp.int32)
    pb = pltpu.bitcast(xb, jnp.int32)
    o_ref[...] = jnp.concatenate([pa, pb], axis=1)


def _pack(tT):
    return pl.pallas_call(
        _pack_body,
        grid=(HALF // CB,),
        in_specs=[
            pl.BlockSpec((64, CB), lambda i: (0, i)),
            pl.BlockSpec((64, CB), lambda i: (0, jnp.minimum(i + 62, 122))),
        ],
        out_specs=pl.BlockSpec((CB // 2, 128), lambda i: (i, 0)),
        out_shape=jax.ShapeDtypeStruct((Q, 128), jnp.int32),
    )(tT, tT)


def _body(sess_hbm, aid_hbm, asz_hbm, stbl_hbm, atbl_hbm, out_hbm,
          sidx_o, aidx_o, sidx_p, aidx_p, asz_v, srows, arows, out_v,
          sem_in, sem_s, sem_a):
    wid = lax.axis_index("c") * 16 + lax.axis_index("s")

    c1 = pltpu.async_copy(sess_hbm.at[wid], sidx_o, sem_in)
    c2 = pltpu.async_copy(aid_hbm.at[wid], aidx_o, sem_in)
    c3 = pltpu.async_copy(asz_hbm.at[wid], asz_v, sem_in)
    c1.wait()
    c2.wait()
    c3.wait()

    def remap(i, _):
        c = i // 8
        l = (i % 8) * 16
        ov = sidx_o[c, pl.ds(l, 16)]
        sidx_p[c, pl.ds(l, 16)] = (ov - jnp.where(
            ov >= HALF, jnp.int32(HALF), jnp.int32(0))) >> 1
        av = aidx_o[c, pl.ds(l, 16)]
        aidx_p[c, pl.ds(l, 16)] = (av - jnp.where(
            av >= HALF, jnp.int32(HALF), jnp.int32(0))) >> 1
        return 0
    lax.fori_loop(0, N_CHUNKS * 8, remap, 0)

    lane = jnp.arange(LANES, dtype=jnp.int32)

    for p in range(N_PASSES):
        copies = []
        for j in range(2):
            c = p * 2 + j
            copies.append(pltpu.async_copy(
                stbl_hbm.at[sidx_p.at[c]],
                srows.at[pl.ds(j * IDX_CHUNK, IDX_CHUNK)], sem_s))
            copies.append(pltpu.async_copy(
                atbl_hbm.at[aidx_p.at[c]],
                arows.at[pl.ds(j * IDX_CHUNK, IDX_CHUNK)], sem_a))
        for c in copies:
            c.wait()

        def group_body(g, _):
            row = g * LANES + lane
            ch = p * 2 + g // 8
            l = (g % 8) * 16
            sv_o = sidx_o[ch, pl.ds(l, 16)]
            av_o = aidx_o[ch, pl.ds(l, 16)]
            so = jnp.where(sv_o >= HALF, jnp.int32(N_FACTORS), jnp.int32(0))
            ao = jnp.where(av_o >= HALF, jnp.int32(N_FACTORS), jnp.int32(0))
            sp = (sv_o & 1) == 1
            ap = (av_o & 1) == 1

            def col_body(f, acc):
                sw = plsc.load_gather(srows, [row, so + f])
                aw = plsc.load_gather(arows, [row, ao + f])
                se, sod = plsc.unpack(plsc.bitcast(sw, jnp.bfloat16),
                                      format=plsc.PackFormat.INTERLEAVED)
                ae, aod = plsc.unpack(plsc.bitcast(aw, jnp.bfloat16),
                                      format=plsc.PackFormat.INTERLEAVED)
                s = jnp.where(sp, sod, se)
                a = jnp.where(ap, aod, ae)
                return acc + s * a

            acc = lax.fori_loop(0, N_FACTORS, col_body,
                                jnp.zeros((LANES,), jnp.float32))
            scale = asz_v[pl.ds(p * B_PER_PASS + g * LANES, LANES)]
            out_v[pl.ds(p * B_PER_PASS + g * LANES, LANES)] = acc * scale
            return 0

        lax.fori_loop(0, GROUPS_PER_PASS, group_body, 0)

    pltpu.sync_copy(out_v, out_hbm.at[wid])


def kernel(session, aid, aid_size, session_table, aid_table):
    mesh = plsc.VectorSubcoreMesh(core_axis_name="c", subcore_axis_name="s")
    k = pl.kernel(
        _body,
        out_type=jax.ShapeDtypeStruct((NUM_WORKERS, B_PER_W), jnp.float32),
        mesh=mesh,
        compiler_params=pltpu.CompilerParams(
            needs_layout_passes=False, use_tc_tiling_on_sc=False),
        scratch_types=[
            pltpu.VMEM((N_CHUNKS, IDX_CHUNK), jnp.int32),
            pltpu.VMEM((N_CHUNKS, IDX_CHUNK), jnp.int32),
            pltpu.VMEM((N_CHUNKS, IDX_CHUNK), jnp.int32),
            pltpu.VMEM((N_CHUNKS, IDX_CHUNK), jnp.int32),
            pltpu.VMEM((B_PER_W,), jnp.float32),
            pltpu.VMEM((B_PER_PASS, 128), jnp.int32),
            pltpu.VMEM((B_PER_PASS, 128), jnp.int32),
            pltpu.VMEM((B_PER_W,), jnp.float32),
            pltpu.SemaphoreType.DMA,
            pltpu.SemaphoreType.DMA,
            pltpu.SemaphoreType.DMA,
        ],
    )
    sess = session.astype(jnp.int32).reshape(NUM_WORKERS, N_CHUNKS, IDX_CHUNK)
    aidr = aid.astype(jnp.int32).reshape(NUM_WORKERS, N_CHUNKS, IDX_CHUNK)
    aszr = aid_size.reshape(NUM_WORKERS, B_PER_W)
    ps = _pack(jnp.swapaxes(session_table, 0, 1))
    pa = _pack(jnp.swapaxes(aid_table, 0, 1))
    out = k(sess, aidr, aszr, ps, pa)
    return out.reshape(BATCH)


# speedup vs baseline: 10.0525x; 1.1104x over previous
import jax
import jax.numpy as jnp
from jax import lax
from jax.experimental import pallas as pl
from jax.experimental.pallas import tpu as pltpu
from jax.experimental.pallas import tpu_sc as plsc

N_FACTORS = 64
BATCH = 16384
NUM_WORKERS = 32
B_PER_W = BATCH // NUM_WORKERS
IDX_CHUNK = 128
N_CHUNKS = B_PER_W // IDX_CHUNK
LANES = 16
N_PASSES = 2
B_PER_PASS = B_PER_W // N_PASSES
GROUPS_PER_PASS = B_PER_PASS // LANES

CB = 16384
HALF = 507904
Q = HALF // 2
N_BLOCKS = 123


def _pack_body(a_ref, b_ref, o_ref):
    xa = jnp.swapaxes(a_ref[...].astype(jnp.bfloat16), 0, 1)
    xb = jnp.swapaxes(b_ref[...].astype(jnp.bfloat16), 0, 1)
    pa = pltpu.bitcast(xa, jnp.int32)
    pb = pltpu.bitcast(xb, jnp.int32)
    o_ref[...] = jnp.concatenate([pa, pb], axis=1)


def _pack(tT):
    return pl.pallas_call(
        _pack_body,
        grid=(HALF // CB,),
        in_specs=[
            pl.BlockSpec((64, CB), lambda i: (0, i)),
            pl.BlockSpec((64, CB), lambda i: (0, jnp.minimum(i + 31, 61))),
        ],
        out_specs=pl.BlockSpec((CB // 2, 128), lambda i: (i, 0)),
        out_shape=jax.ShapeDtypeStruct((Q, 128), jnp.int32),
    )(tT, tT)


def _body(sess_hbm, aid_hbm, asz_hbm, stbl_hbm, atbl_hbm, out_hbm,
          sidx_o, aidx_o, sidx_p, aidx_p, asz_v, srows, arows, out_v,
          sem_in, sem_s, sem_a):
    wid = lax.axis_index("c") * 16 + lax.axis_index("s")

    c1 = pltpu.async_copy(sess_hbm.at[wid], sidx_o, sem_in)
    c2 = pltpu.async_copy(aid_hbm.at[wid], aidx_o, sem_in)
    c3 = pltpu.async_copy(asz_hbm.at[wid], asz_v, sem_in)
    c1.wait()
    c2.wait()
    c3.wait()

    def remap(i, _):
        c = i // 8
        l = (i % 8) * 16
        ov = sidx_o[c, pl.ds(l, 16)]
        sidx_p[c, pl.ds(l, 16)] = (ov - jnp.where(
            ov >= HALF, jnp.int32(HALF), jnp.int32(0))) >> 1
        av = aidx_o[c, pl.ds(l, 16)]
        aidx_p[c, pl.ds(l, 16)] = (av - jnp.where(
            av >= HALF, jnp.int32(HALF), jnp.int32(0))) >> 1
        return 0
    lax.fori_loop(0, N_CHUNKS * 8, remap, 0)

    lane = jnp.arange(LANES, dtype=jnp.int32)

    for p in range(N_PASSES):
        copies = []
        for j in range(2):
            c = p * 2 + j
            copies.append(pltpu.async_copy(
                stbl_hbm.at[sidx_p.at[c]],
                srows.at[pl.ds(j * IDX_CHUNK, IDX_CHUNK)], sem_s))
            copies.append(pltpu.async_copy(
                atbl_hbm.at[aidx_p.at[c]],
                arows.at[pl.ds(j * IDX_CHUNK, IDX_CHUNK)], sem_a))
        for c in copies:
            c.wait()

        def group_body(g, _):
            row = g * LANES + lane
            ch = p * 2 + g // 8
            l = (g % 8) * 16
            sv_o = sidx_o[ch, pl.ds(l, 16)]
            av_o = aidx_o[ch, pl.ds(l, 16)]
            so = jnp.where(sv_o >= HALF, jnp.int32(N_FACTORS), jnp.int32(0))
            ao = jnp.where(av_o >= HALF, jnp.int32(N_FACTORS), jnp.int32(0))
            sp = (sv_o & 1) == 1
            ap = (av_o & 1) == 1

            def col_body(f, acc):
                sw = plsc.load_gather(srows, [row, so + f])
                aw = plsc.load_gather(arows, [row, ao + f])
                se, sod = plsc.unpack(plsc.bitcast(sw, jnp.bfloat16),
                                      format=plsc.PackFormat.INTERLEAVED)
                ae, aod = plsc.unpack(plsc.bitcast(aw, jnp.bfloat16),
                                      format=plsc.PackFormat.INTERLEAVED)
                s = jnp.where(sp, sod, se)
                a = jnp.where(ap, aod, ae)
                return acc + s * a

            acc = lax.fori_loop(0, N_FACTORS, col_body,
                                jnp.zeros((LANES,), jnp.float32))
            scale = asz_v[pl.ds(p * B_PER_PASS + g * LANES, LANES)]
            out_v[pl.ds(p * B_PER_PASS + g * LANES, LANES)] = acc * scale
            return 0

        lax.fori_loop(0, GROUPS_PER_PASS, group_body, 0)

    pltpu.sync_copy(out_v, out_hbm.at[wid])


def kernel(session, aid, aid_size, session_table, aid_table):
    mesh = plsc.VectorSubcoreMesh(core_axis_name="c", subcore_axis_name="s")
    k = pl.kernel(
        _body,
        out_type=jax.ShapeDtypeStruct((NUM_WORKERS, B_PER_W), jnp.float32),
        mesh=mesh,
        compiler_params=pltpu.CompilerParams(
            needs_layout_passes=False, use_tc_tiling_on_sc=False),
        scratch_types=[
            pltpu.VMEM((N_CHUNKS, IDX_CHUNK), jnp.int32),
            pltpu.VMEM((N_CHUNKS, IDX_CHUNK), jnp.int32),
            pltpu.VMEM((N_CHUNKS, IDX_CHUNK), jnp.int32),
            pltpu.VMEM((N_CHUNKS, IDX_CHUNK), jnp.int32),
            pltpu.VMEM((B_PER_W,), jnp.float32),
            pltpu.VMEM((B_PER_PASS, 128), jnp.int32),
            pltpu.VMEM((B_PER_PASS, 128), jnp.int32),
            pltpu.VMEM((B_PER_W,), jnp.float32),
            pltpu.SemaphoreType.DMA,
            pltpu.SemaphoreType.DMA,
            pltpu.SemaphoreType.DMA,
        ],
    )
    sess = session.astype(jnp.int32).reshape(NUM_WORKERS, N_CHUNKS, IDX_CHUNK)
    aidr = aid.astype(jnp.int32).reshape(NUM_WORKERS, N_CHUNKS, IDX_CHUNK)
    aszr = aid_size.reshape(NUM_WORKERS, B_PER_W)
    ps = _pack(jnp.swapaxes(session_table, 0, 1))
    pa = _pack(jnp.swapaxes(aid_table, 0, 1))
    out = k(sess, aidr, aszr, ps, pa)
    return out.reshape(BATCH)
